# baseline (speedup 1.0000x reference)
import functools

import jax
import jax.numpy as jnp
from jax import lax
from jax.experimental import pallas as pl
from jax.experimental.pallas import tpu as pltpu
from jax.experimental.pallas import tpu_sc as plsc

S = 1024
N = 50
D = 128
SEQ = 3 * N - 1
NC = 2
NS = 16
NW = NC * NS

CK = 128
CPC = S // CK
EX_CHUNKS = 2 * N * CPC
LB_REAL = (N - 1) * CPC
EX_PER_W = EX_CHUNKS // NW
LB_PER_W = 13
LB_PAD = NW * LB_PER_W - LB_REAL
EX_NBUF = 5


def _sc_gather_kernel():
    mesh = plsc.VectorSubcoreMesh(core_axis_name="c", subcore_axis_name="s")

    @functools.partial(
        pl.kernel,
        mesh=mesh,
        out_type=jax.ShapeDtypeStruct((SEQ * S, D), jnp.float32),
        scratch_types=(
            [pltpu.VMEM((CK, D), jnp.float32) for _ in range(EX_NBUF)]
            + [
                pltpu.VMEM((EX_PER_W, CK), jnp.int32),
                pltpu.VMEM((LB_PER_W, CK), jnp.int32),
            ]
            + [pltpu.SemaphoreType.DMA for _ in range(2 * EX_NBUF)]
        ),
    )
    def k(emb, lemb, ex_src, lb_src, out, *scratch):
        bufs = scratch[:EX_NBUF]
        sidx_e, sidx_l = scratch[EX_NBUF:EX_NBUF + 2]
        sem_g = scratch[EX_NBUF + 2:EX_NBUF + 2 + EX_NBUF]
        sem_s = scratch[EX_NBUF + 2 + EX_NBUF:]

        wid = lax.axis_index("s") * NC + lax.axis_index("c")

        pltpu.sync_copy(ex_src.at[wid], sidx_e)
        pltpu.sync_copy(lb_src.at[wid], sidx_l)

        def ex_base(j):
            g = wid * EX_PER_W + j
            col = g >> 3
            seg = g & 7
            t = 3 * (col >> 1) + (col & 1)
            return t * S + seg * CK

        def lb_base(i):
            g = wid * LB_PER_W + i
            g = jnp.where(g < LB_REAL, g, g - LB_REAL)
            col = g >> 3
            seg = g & 7
            t = 3 * col + 2
            return t * S + seg * CK

        def start_gather(tbl, sidx, j, b):
            pltpu.async_copy(tbl.at[sidx.at[j]], bufs[b], sem_g[b])

        def wait_gather(tbl, sidx, j, b):
            pltpu.make_async_copy(tbl.at[sidx.at[j]], bufs[b],
                                  sem_g[b]).wait()

        def start_scatter(base, b):
            pltpu.async_copy(bufs[b], out.at[pl.ds(base, CK)], sem_s[b])

        def wait_scatter(base, b):
            pltpu.make_async_copy(bufs[b], out.at[pl.ds(base, CK)],
                                  sem_s[b]).wait()

        for b in range(EX_NBUF):
            start_gather(emb, sidx_e, b, b)

        def lap(t, carry):
            for b in range(EX_NBUF):
                j = t * EX_NBUF + b
                wait_gather(emb, sidx_e, j, b)
                start_scatter(ex_base(j), b)
            for b in range(EX_NBUF):
                j = t * EX_NBUF + b
                wait_scatter(ex_base(j), b)
                start_gather(emb, sidx_e, j + EX_NBUF, b)
            return carry

        laps = EX_PER_W // EX_NBUF
        lax.fori_loop(0, laps - 1, lap, 0, unroll=False)
        for b in range(EX_NBUF):
            j = (laps - 1) * EX_NBUF + b
            wait_gather(emb, sidx_e, j, b)
            start_scatter(ex_base(j), b)
        for b in range(EX_NBUF):
            wait_scatter(ex_base((laps - 1) * EX_NBUF + b), b)

        for i in range(min(EX_NBUF, LB_PER_W)):
            start_gather(lemb, sidx_l, i, i)
        for lo in range(0, LB_PER_W, EX_NBUF):
            group = range(lo, min(lo + EX_NBUF, LB_PER_W))
            for i in group:
                wait_gather(lemb, sidx_l, i, i % EX_NBUF)
                start_scatter(lb_base(i), i % EX_NBUF)
            for i in group:
                nxt = i + EX_NBUF
                if nxt < LB_PER_W:
                    wait_scatter(lb_base(i), i % EX_NBUF)
                    start_gather(lemb, sidx_l, nxt, i % EX_NBUF)
        for i in range(LB_PER_W - EX_NBUF, LB_PER_W):
            wait_scatter(lb_base(i), i % EX_NBUF)

    return k


_KERNEL = _sc_gather_kernel()


def kernel(examples, labels, embeddings, label_embeddings):
    ex_src = examples.T.reshape(NW, EX_PER_W, CK)
    lb_flat = labels[:, : N - 1].T.reshape(LB_REAL, CK)
    lb_src = jnp.concatenate([lb_flat, lb_flat[:LB_PAD]]
                             ).reshape(NW, LB_PER_W, CK)

    out = _KERNEL(embeddings, label_embeddings, ex_src, lb_src)
    return jnp.swapaxes(out.reshape(SEQ, S, D), 0, 1)


# speedup vs baseline: 1.2824x; 1.2824x over previous
import functools

import jax
import jax.numpy as jnp
from jax import lax
from jax.experimental import pallas as pl
from jax.experimental.pallas import tpu as pltpu
from jax.experimental.pallas import tpu_sc as plsc

S = 1024
N = 50
D = 128
SEQ = 3 * N - 1
NC = 2
NS = 16
NW = NC * NS

CK = 128
CPC = S // CK
EX_CHUNKS = 2 * N * CPC
LB_REAL = (N - 1) * CPC
EX_PER_W = EX_CHUNKS // NW
LB_PER_W = 13
LB_PAD = NW * LB_PER_W - LB_REAL
LV = 1000
EX_NBUF = 5


def _sc_gather_kernel():
    mesh = plsc.VectorSubcoreMesh(core_axis_name="c", subcore_axis_name="s")

    @functools.partial(
        pl.kernel,
        mesh=mesh,
        out_type=jax.ShapeDtypeStruct((SEQ * S, D), jnp.float32),
        scratch_types=(
            [pltpu.VMEM((CK, D), jnp.float32) for _ in range(EX_NBUF)]
            + [
                pltpu.VMEM((EX_PER_W, CK), jnp.int32),
                pltpu.VMEM((LB_PER_W, CK), jnp.int32),
                pltpu.VMEM_SHARED((LV, D), jnp.float32),
            ]
            + [pltpu.SemaphoreType.DMA for _ in range(2 * EX_NBUF + 1)]
        ),
    )
    def k(emb, lemb, ex_src, lb_src, out, *scratch):
        bufs = scratch[:EX_NBUF]
        sidx_e, sidx_l, ltab = scratch[EX_NBUF:EX_NBUF + 3]
        sem_g = scratch[EX_NBUF + 3:EX_NBUF + 3 + EX_NBUF]
        sem_s = scratch[EX_NBUF + 3 + EX_NBUF:EX_NBUF + 3 + 2 * EX_NBUF]
        sem_t = scratch[-1]

        sid = lax.axis_index("s")
        wid = sid * NC + lax.axis_index("c")

        @pl.when(sid == 0)
        def _():
            pltpu.async_copy(lemb, ltab, sem_t)

        pltpu.sync_copy(ex_src.at[wid], sidx_e)
        pltpu.sync_copy(lb_src.at[wid], sidx_l)

        def ex_base(j):
            g = wid * EX_PER_W + j
            col = g >> 3
            seg = g & 7
            t = 3 * (col >> 1) + (col & 1)
            return t * S + seg * CK

        def lb_base(i):
            g = wid * LB_PER_W + i
            g = jnp.where(g < LB_REAL, g, g - LB_REAL)
            col = g >> 3
            seg = g & 7
            t = 3 * col + 2
            return t * S + seg * CK

        def start_gather(tbl, sidx, j, b):
            pltpu.async_copy(tbl.at[sidx.at[j]], bufs[b], sem_g[b])

        def wait_gather(tbl, sidx, j, b):
            pltpu.make_async_copy(tbl.at[sidx.at[j]], bufs[b],
                                  sem_g[b]).wait()

        def start_scatter(base, b):
            pltpu.async_copy(bufs[b], out.at[pl.ds(base, CK)], sem_s[b])

        def wait_scatter(base, b):
            pltpu.make_async_copy(bufs[b], out.at[pl.ds(base, CK)],
                                  sem_s[b]).wait()

        for b in range(EX_NBUF):
            start_gather(emb, sidx_e, b, b)

        def lap(t, carry):
            for b in range(EX_NBUF):
                j = t * EX_NBUF + b
                wait_gather(emb, sidx_e, j, b)
                start_scatter(ex_base(j), b)
            for b in range(EX_NBUF):
                j = t * EX_NBUF + b
                wait_scatter(ex_base(j), b)
                start_gather(emb, sidx_e, j + EX_NBUF, b)
            return carry

        laps = EX_PER_W // EX_NBUF
        lax.fori_loop(0, laps - 1, lap, 0, unroll=False)
        for b in range(EX_NBUF):
            j = (laps - 1) * EX_NBUF + b
            wait_gather(emb, sidx_e, j, b)
            start_scatter(ex_base(j), b)
        for b in range(EX_NBUF):
            wait_scatter(ex_base((laps - 1) * EX_NBUF + b), b)

        @pl.when(sid == 0)
        def _():
            pltpu.make_async_copy(lemb, ltab, sem_t).wait()

        plsc.subcore_barrier()

        for i in range(min(EX_NBUF, LB_PER_W)):
            start_gather(ltab, sidx_l, i, i)
        for lo in range(0, LB_PER_W, EX_NBUF):
            group = range(lo, min(lo + EX_NBUF, LB_PER_W))
            for i in group:
                wait_gather(ltab, sidx_l, i, i % EX_NBUF)
                start_scatter(lb_base(i), i % EX_NBUF)
            for i in group:
                nxt = i + EX_NBUF
                if nxt < LB_PER_W:
                    wait_scatter(lb_base(i), i % EX_NBUF)
                    start_gather(ltab, sidx_l, nxt, i % EX_NBUF)
        for i in range(LB_PER_W - EX_NBUF, LB_PER_W):
            wait_scatter(lb_base(i), i % EX_NBUF)

    return k


_KERNEL = _sc_gather_kernel()


def kernel(examples, labels, embeddings, label_embeddings):
    ex_src = examples.T.reshape(NW, EX_PER_W, CK)
    lb_flat = labels[:, : N - 1].T.reshape(LB_REAL, CK)
    lb_src = jnp.concatenate([lb_flat, lb_flat[:LB_PAD]]
                             ).reshape(NW, LB_PER_W, CK)

    out = _KERNEL(embeddings, label_embeddings, ex_src, lb_src)
    return jnp.swapaxes(out.reshape(SEQ, S, D), 0, 1)
